# baseline (speedup 1.0000x reference)
import functools

import jax
import jax.numpy as jnp
from jax import lax
from jax.experimental import pallas as pl
from jax.experimental.pallas import tpu as pltpu
from jax.experimental.pallas import tpu_sc as plsc

B = 128
BEAM = 4
VOCAB = 100000
ROWS = B * BEAM
NEG = -3.0e38

NW = 32
ROWS_W = ROWS // NW
SEG = 2000
NSEG = VOCAB // SEG
NVEC_SEG = SEG // 16
NVEC = VOCAB // 16
CAND_CAP = 2048
UNROLL = 5


def _lse_kernel(m_ref, s_ref, lse_ref):
    m = m_ref[...]
    s = s_ref[...]
    mr = jnp.max(m, axis=1, keepdims=True)
    sr = jnp.sum(s * jnp.exp(m - mr), axis=1, keepdims=True)
    lse_ref[...] = mr + jnp.log(sr)


def _merge_kernel(v_ref, t_ref, bs_ref, lse_ref, os_ref, ot_ref, ob_ref):
    s = v_ref[...] + bs_ref[...] - lse_ref[...]
    t = t_ref[...]
    slot = jax.lax.broadcasted_iota(jnp.int32, s.shape, 1)
    ss, tt, bb = [], [], []
    y = s
    for _ in range(4):
        v = jnp.max(y, axis=1, keepdims=True)
        sl = jnp.min(jnp.where(y == v, slot, 16), axis=1, keepdims=True)
        tok = jnp.max(jnp.where(slot == sl, t, -1), axis=1, keepdims=True)
        ss.append(v)
        tt.append(tok)
        bb.append(sl // 4)
        y = jnp.where(slot == sl, NEG, y)
    os_ref[...] = jnp.concatenate(ss, axis=1)
    ot_ref[...] = jnp.concatenate(tt, axis=1)
    ob_ref[...] = jnp.concatenate(bb, axis=1)


def _sc_topk_body(x_hbm, vals_hbm, idx_hbm, mlan_hbm, slan_hbm,
                  row_v, segmax_v, cval_v, cidx_v,
                  ov_v, oi_v, om_v, os_v, sem):
    wid = lax.axis_index("s") * 2 + lax.axis_index("c")
    base_row = wid * ROWS_W
    lane = lax.broadcasted_iota(jnp.int32, (16,), 0)
    negv = jnp.full((16,), NEG, jnp.float32)
    zerov = jnp.zeros((16,), jnp.float32)

    def do_row(rl, carry):
        row = base_row + rl
        pltpu.sync_copy(x_hbm.at[row], row_v)

        def seg_body(sg, carry):
            def vblk(jb, accs):
                base = sg * SEG + jb * (16 * UNROLL)
                return tuple(
                    jnp.maximum(a, row_v[pl.ds(base + u * 16, 16)])
                    for u, a in enumerate(accs))
            accs = lax.fori_loop(0, NVEC_SEG // UNROLL, vblk, (negv,) * UNROLL)
            m01 = jnp.maximum(accs[0], accs[1])
            m23 = jnp.maximum(accs[2], accs[3])
            segmax_v[sg] = jnp.maximum(jnp.maximum(m01, m23), accs[4])
            return carry
        lax.fori_loop(0, NSEG, seg_body, 0)

        def mrow_body(sg, acc):
            return jnp.maximum(acc, segmax_v[sg])
        mrow = lax.fori_loop(0, NSEG, mrow_body, negv)

        def sblk(jb, accs):
            base = jb * (16 * UNROLL)
            return tuple(
                a + jnp.exp(row_v[pl.ds(base + u * 16, 16)] - mrow)
                for u, a in enumerate(accs))
        saccs = lax.fori_loop(0, NVEC // UNROLL, sblk, (zerov,) * UNROLL)
        srow = (saccs[0] + saccs[1]) + (saccs[2] + saccs[3]) + saccs[4]

        def tau_seg(sg, ts):
            t0, t1, t2, t3 = ts
            v = segmax_v[sg]
            h0 = jnp.maximum(t0, v); l0 = jnp.minimum(t0, v)
            h1 = jnp.maximum(t1, l0); l1 = jnp.minimum(t1, l0)
            h2 = jnp.maximum(t2, l1); l2 = jnp.minimum(t2, l1)
            h3 = jnp.maximum(t3, l2)
            return (h0, h1, h2, h3)
        t0, t1, t2, t3 = lax.fori_loop(
            0, NSEG, tau_seg, (negv, negv, negv, negv))
        tau = jnp.float32(0)
        for _ in range(4):
            m01 = jnp.maximum(jnp.maximum(t0, t1), jnp.maximum(t2, t3))
            tau = jnp.max(m01)
            t0 = jnp.where(t0 == tau, negv, t0)
            t1 = jnp.where(t1 == tau, negv, t1)
            t2 = jnp.where(t2 == tau, negv, t2)
            t3 = jnp.where(t3 == tau, negv, t3)

        def seg_collect(sg, off):
            trig = jnp.max(segmax_v[sg]) >= tau

            def yes(off):
                def body(j, off):
                    v = row_v[pl.ds(sg * SEG + j * 16, 16)]
                    iv = sg * SEG + j * 16 + lane
                    msk = v >= tau
                    o = jnp.minimum(off, CAND_CAP)
                    plsc.store_compressed(cval_v.at[pl.ds(o, 16)], v, mask=msk)
                    plsc.store_compressed(cidx_v.at[pl.ds(o, 16)], iv, mask=msk)
                    cnt = plsc.all_reduce_population_count(msk)
                    return off + cnt[0]
                return lax.fori_loop(0, NVEC_SEG, body, off)

            return lax.cond(trig, yes, lambda off: off, off)
        ncand = lax.fori_loop(0, NSEG, seg_collect, jnp.int32(0))
        ncand = jnp.minimum(ncand, CAND_CAP)
        nvec = (ncand + 15) // 16

        found_v = []
        found_i = []
        for _ in range(4):
            def scan_body(j, st):
                bv, bi = st
                v = cval_v[pl.ds(j * 16, 16)]
                iv = cidx_v[pl.ds(j * 16, 16)]
                ok = (j * 16 + lane) < ncand
                for e in found_i:
                    ok = ok & (iv != e)
                v = jnp.where(ok, v, negv)
                gt = v > bv
                eq = (v == bv) & (iv < bi)
                take = gt | eq
                return (jnp.where(take, v, bv), jnp.where(take, iv, bi))
            bv, bi = lax.fori_loop(
                0, nvec, scan_body,
                (negv, jnp.full((16,), VOCAB, jnp.int32)))
            vm = jnp.max(bv)
            im = jnp.min(jnp.where(bv == vm, bi, VOCAB))
            found_v.append(vm)
            found_i.append(im)

        ov = negv
        oi = jnp.zeros((16,), jnp.int32)
        for k in range(4):
            ov = jnp.where(lane == k, found_v[k], ov)
            oi = jnp.where(lane == k, found_i[k], oi)
        ov_v[rl] = ov
        oi_v[rl] = oi
        om_v[rl] = mrow
        os_v[rl] = srow
        return carry

    lax.fori_loop(0, ROWS_W, do_row, 0)
    pltpu.sync_copy(ov_v, vals_hbm.at[pl.ds(base_row, ROWS_W)])
    pltpu.sync_copy(oi_v, idx_hbm.at[pl.ds(base_row, ROWS_W)])
    pltpu.sync_copy(om_v, mlan_hbm.at[pl.ds(base_row, ROWS_W)])
    pltpu.sync_copy(os_v, slan_hbm.at[pl.ds(base_row, ROWS_W)])


@jax.jit
def kernel(logits, beam_scores):
    b, beam, vocab = logits.shape
    rows = b * beam

    sc_topk = functools.partial(
        pl.kernel,
        mesh=plsc.VectorSubcoreMesh(core_axis_name="c", subcore_axis_name="s"),
        compiler_params=pltpu.CompilerParams(
            needs_layout_passes=False, use_tc_tiling_on_sc=True),
        out_type=[
            jax.ShapeDtypeStruct((rows, 16), jnp.float32),
            jax.ShapeDtypeStruct((rows, 16), jnp.int32),
            jax.ShapeDtypeStruct((rows, 16), jnp.float32),
            jax.ShapeDtypeStruct((rows, 16), jnp.float32),
        ],
        scratch_types=[
            pltpu.VMEM((vocab,), jnp.float32),
            pltpu.VMEM((NSEG, 16), jnp.float32),
            pltpu.VMEM((CAND_CAP + 16,), jnp.float32),
            pltpu.VMEM((CAND_CAP + 16,), jnp.int32),
            pltpu.VMEM((ROWS_W, 16), jnp.float32),
            pltpu.VMEM((ROWS_W, 16), jnp.int32),
            pltpu.VMEM((ROWS_W, 16), jnp.float32),
            pltpu.VMEM((ROWS_W, 16), jnp.float32),
            pltpu.SemaphoreType.DMA,
        ],
    )(_sc_topk_body)
    vals, idx, mlan, slan = sc_topk(logits.reshape(rows, vocab))

    lse = pl.pallas_call(
        _lse_kernel,
        out_shape=jax.ShapeDtypeStruct((rows, 1), jnp.float32),
    )(mlan, slan)

    v16 = vals[:, :4].reshape(b, 16)
    t16 = idx[:, :4].reshape(b, 16)
    bs16 = jnp.repeat(beam_scores, 4, axis=1)
    lse16 = jnp.repeat(lse.reshape(b, beam), 4, axis=1)

    os_, ot, ob = pl.pallas_call(
        _merge_kernel,
        out_shape=[
            jax.ShapeDtypeStruct((b, 4), jnp.float32),
            jax.ShapeDtypeStruct((b, 4), jnp.int32),
            jax.ShapeDtypeStruct((b, 4), jnp.int32),
        ],
    )(v16, t16, bs16, lse16)

    return os_, ot, ob


# speedup vs baseline: 1.7941x; 1.7941x over previous
import functools

import jax
import jax.numpy as jnp
from jax import lax
from jax.experimental import pallas as pl
from jax.experimental.pallas import tpu as pltpu
from jax.experimental.pallas import tpu_sc as plsc

B = 128
BEAM = 4
VOCAB = 100000
NEG = -3.0e38
INTBIG = 2 ** 30

CHUNK = 400
NCH = VOCAB // CHUNK
NSTR = 8
NLG = 8

LSE_BLK = VOCAB // 20


def _lse_part_kernel(x_ref, m_ref, s_ref):
    x = x_ref[0]
    mx = jnp.max(x, axis=0, keepdims=True)
    s = jnp.sum(jnp.exp(x - mx), axis=0, keepdims=True)
    m_ref[...] = mx[None, None]
    s_ref[...] = s[None, None]


def _merge_kernel(cv_ref, ct_ref, mp_ref, sp_ref, bs_ref,
                  os_ref, ot_ref, ob_ref):
    cv = cv_ref[...]
    ct = ct_ref[...]
    mp = mp_ref[...]
    sp = sp_ref[...]
    bs = bs_ref[...]

    grp = jax.lax.broadcasted_iota(jnp.int32, mp.shape, 1) // 20
    slotbeam = jax.lax.broadcasted_iota(jnp.int32, cv.shape, 1) // 32

    adj = jnp.zeros_like(cv)
    for m in range(BEAM):
        sel = grp == m
        mb = jnp.max(jnp.where(sel, mp, NEG), axis=1, keepdims=True)
        sb = jnp.sum(jnp.where(sel, sp * jnp.exp(mp - mb), 0.0),
                     axis=1, keepdims=True)
        lse = mb + jnp.log(sb)
        adj = adj + jnp.where(slotbeam == m, bs[:, m:m + 1] - lse, 0.0)

    y = cv + adj
    fl = slotbeam * VOCAB + ct
    ss, tt, bb = [], [], []
    for _ in range(4):
        v = jnp.max(y, axis=1, keepdims=True)
        flb = jnp.min(jnp.where(y == v, fl, INTBIG), axis=1, keepdims=True)
        ss.append(v)
        tt.append(flb % VOCAB)
        bb.append(flb // VOCAB)
        y = jnp.where(fl == flb, NEG, y)
    os_ref[...] = jnp.concatenate(ss, axis=1)
    ot_ref[...] = jnp.concatenate(tt, axis=1)
    ob_ref[...] = jnp.concatenate(bb, axis=1)


def _sc_topk_body(x_hbm, vals_hbm, idx_hbm, buf_v, iv_v, stv_v, sti_v, sem):
    wid = lax.axis_index("s") * 2 + lax.axis_index("c")
    m = wid // NSTR
    j = wid - m * NSTR
    nk = (NCH - 1 - j) // NSTR + 1
    lane = lax.broadcasted_iota(jnp.int32, (16,), 0)
    negv = jnp.full((16,), NEG, jnp.float32)

    def chunk_src(k):
        return x_hbm.at[m, pl.ds(k * CHUNK, CHUNK)]

    pltpu.async_copy(chunk_src(j), buf_v.at[0], sem)

    def chunk_body(c, carry):
        mm, tq = carry
        k = j + NSTR * c
        sel = lax.rem(c, 2)
        knext = jnp.minimum(k + NSTR, NCH - 1)
        pltpu.async_copy(chunk_src(knext), buf_v.at[1 - sel], sem)
        pltpu.make_async_copy(chunk_src(k), buf_v.at[sel], sem).wait()

        def v_body(v, carry):
            mm, tq = carry
            vabs = k * CHUNK + v
            vs = [buf_v[sel, v, pl.ds(lg * 16, 16)] for lg in range(NLG)]
            mm = tuple(jnp.maximum(mm[lg], vs[lg]) for lg in range(NLG))
            tms = [vs[lg] > tq[lg * 4 + 3] for lg in range(NLG)]
            t01 = tms[0] | tms[1]
            t23 = tms[2] | tms[3]
            t45 = tms[4] | tms[5]
            t67 = tms[6] | tms[7]
            tor = (t01 | t23) | (t45 | t67)
            cnt = plsc.all_reduce_population_count(tor)[0]

            def slow(tq):
                out = list(tq)
                for lg in range(NLG):
                    clg = vs[lg] > tq[lg * 4 + 3]
                    cl = plsc.all_reduce_population_count(clg)[0]

                    def ins(q4, lg=lg):
                        q = list(q4)
                        t = vs[lg]
                        ti = lane * 0 + vabs
                        for r in range(4):
                            iv = iv_v[lg, r]
                            cc = t > q[r]
                            nv = jnp.where(cc, t, q[r])
                            ni = jnp.where(cc, ti, iv)
                            t = jnp.where(cc, q[r], t)
                            ti = jnp.where(cc, iv, ti)
                            q[r] = nv
                            iv_v[lg, r] = ni
                        return tuple(q)

                    q4 = lax.cond(cl > 0, ins, lambda q4: q4,
                                  tuple(out[lg * 4:lg * 4 + 4]))
                    out[lg * 4:lg * 4 + 4] = list(q4)
                return tuple(out)

            tq = lax.cond(cnt > 0, slow, lambda tq: tq, tq)
            return (mm, tq)

        return lax.fori_loop(0, CHUNK, v_body, (mm, tq))

    mm0 = (negv,) * NLG
    tq0 = (negv,) * (NLG * 4)
    mm, tq = lax.fori_loop(0, nk, chunk_body, (mm0, tq0))
    pltpu.make_async_copy(chunk_src(0), buf_v.at[0], sem).wait()

    for r in range(4):
        for lg in range(NLG):
            stv_v[r, pl.ds(lg * 16, 16)] = tq[lg * 4 + r]
            sti_v[r, pl.ds(lg * 16, 16)] = iv_v[lg, r]
    pltpu.sync_copy(stv_v, vals_hbm.at[wid])
    pltpu.sync_copy(sti_v, idx_hbm.at[wid])


@jax.jit
def kernel(logits, beam_scores):
    b, beam, vocab = logits.shape
    xT = jnp.transpose(logits, (1, 2, 0))

    sc_topk = functools.partial(
        pl.kernel,
        mesh=plsc.VectorSubcoreMesh(core_axis_name="c", subcore_axis_name="s"),
        compiler_params=pltpu.CompilerParams(
            needs_layout_passes=False, use_tc_tiling_on_sc=True),
        out_type=[
            jax.ShapeDtypeStruct((32, 4, b), jnp.float32),
            jax.ShapeDtypeStruct((32, 4, b), jnp.int32),
        ],
        scratch_types=[
            pltpu.VMEM((2, CHUNK, b), jnp.float32),
            pltpu.VMEM((NLG, 4, 16), jnp.int32),
            pltpu.VMEM((4, b), jnp.float32),
            pltpu.VMEM((4, b), jnp.int32),
            pltpu.SemaphoreType.DMA,
        ],
    )(_sc_topk_body)
    cvals, cidx = sc_topk(xT)

    mpart, spart = pl.pallas_call(
        _lse_part_kernel,
        grid=(beam, vocab // LSE_BLK),
        in_specs=[pl.BlockSpec((1, LSE_BLK, b), lambda i, j: (i, j, 0))],
        out_specs=[
            pl.BlockSpec((1, 1, 1, b), lambda i, j: (i, j, 0, 0)),
            pl.BlockSpec((1, 1, 1, b), lambda i, j: (i, j, 0, 0)),
        ],
        out_shape=[
            jax.ShapeDtypeStruct((beam, vocab // LSE_BLK, 1, b), jnp.float32),
            jax.ShapeDtypeStruct((beam, vocab // LSE_BLK, 1, b), jnp.float32),
        ],
    )(xT)

    cv = cvals.transpose(2, 0, 1).reshape(b, 128)
    ct = cidx.transpose(2, 0, 1).reshape(b, 128)
    mp = mpart.reshape(beam, vocab // LSE_BLK, b).transpose(2, 0, 1).reshape(
        b, beam * (vocab // LSE_BLK))
    sp = spart.reshape(beam, vocab // LSE_BLK, b).transpose(2, 0, 1).reshape(
        b, beam * (vocab // LSE_BLK))

    os_, ot, ob = pl.pallas_call(
        _merge_kernel,
        out_shape=[
            jax.ShapeDtypeStruct((b, 4), jnp.float32),
            jax.ShapeDtypeStruct((b, 4), jnp.int32),
            jax.ShapeDtypeStruct((b, 4), jnp.int32),
        ],
    )(cv, ct, mp, sp, beam_scores)

    return os_, ot, ob
